# baseline (speedup 1.0000x reference)
import functools

import jax
import jax.numpy as jnp
from jax import lax
from jax.experimental import pallas as pl
from jax.experimental.pallas import tpu as pltpu
from jax.experimental.pallas import tpu_sc as plsc

N_NODES = 100000
D = 256
B = 16384

_NC = 2
_NS = 16
_NW = _NC * _NS
_K = 128

_G_BPW = B // _NW
_G_NCH = _G_BPW // _K

_gather_mesh = plsc.VectorSubcoreMesh(core_axis_name="c", subcore_axis_name="s")

_A_ROWS = N_NODES // _NW
_A_SLAB = 3200


@functools.partial(
    pl.kernel,
    out_type=[
        jax.ShapeDtypeStruct((B, D), jnp.float32),
        jax.ShapeDtypeStruct((N_NODES,), jnp.int32),
    ],
    mesh=_gather_mesh,
    scratch_types=[
        pltpu.VMEM((_G_NCH, _K), jnp.int32),
        pltpu.VMEM((B,), jnp.int32),
        pltpu.VMEM((_A_SLAB,), jnp.int32),
        pltpu.VMEM((_K, D), jnp.float32),
        pltpu.VMEM((_K, D), jnp.float32),
        pltpu.SemaphoreType.DMA,
        pltpu.SemaphoreType.DMA,
    ],
    compiler_params=pltpu.CompilerParams(needs_layout_passes=False),
)
def _sc_gather(mem_hbm, ids_hbm, out_hbm, aux_hbm,
               idx_v, allids_v, aux_v, buf0, buf1, sem0, sem1):
    wid = lax.axis_index("s") * _NC + lax.axis_index("c")
    base = wid * _G_BPW
    for c in range(_G_NCH):
        pltpu.sync_copy(ids_hbm.at[pl.ds(base + c * _K, _K)], idx_v.at[c])
    pltpu.sync_copy(ids_hbm, allids_v)

    bufs = (buf0, buf1)
    sems = (sem0, sem1)
    cps = [None, None]
    for c in range(_G_NCH):
        cps[c % 2] = pltpu.async_copy(mem_hbm.at[idx_v.at[c]], bufs[c % 2], sems[c % 2])
        if c >= 1:
            cps[(c - 1) % 2].wait()
            pltpu.sync_copy(bufs[(c - 1) % 2], out_hbm.at[pl.ds(base + (c - 1) * _K, _K)])
    cps[(_G_NCH - 1) % 2].wait()
    pltpu.sync_copy(bufs[(_G_NCH - 1) % 2],
                    out_hbm.at[pl.ds(base + (_G_NCH - 1) * _K, _K)])

    alo = jnp.minimum((wid * _A_ROWS) // 8 * 8, N_NODES - _A_SLAB)
    lane = lax.iota(jnp.int32, 16)
    nb_idx = jnp.minimum(lane + 1, 15)

    def _win_body(i, _):
        ids16 = allids_v[pl.ds(i * 16, 16)]
        ivec = lane + i * 16
        ks, iwin = plsc.sort_key_val(ids16 * 16384 + ivec, ivec)
        sid16 = lax.shift_right_logical(ks, 14)
        nb = sid16.at[nb_idx].get(mode=lax.GatherScatterMode.PROMISE_IN_BOUNDS)
        m = ((lane == 15) | (nb != sid16)) & (sid16 >= alo) & (sid16 < alo + _A_SLAB)
        plsc.store_scatter(aux_v, [jnp.where(m, sid16 - alo, 0)], iwin, mask=m)
        return _

    lax.fori_loop(0, B // 16, _win_body, jnp.int32(0))
    pltpu.sync_copy(aux_v, aux_hbm.at[pl.ds(alo, _A_SLAB)])


_R = 1024


def _gru_body(x_ref, h_ref, wi_ref, wh_ref, bi_ref, bh_ref, out_ref):
    h = h_ref[...]
    gi = jnp.dot(x_ref[...], wi_ref[...], preferred_element_type=jnp.float32) + bi_ref[...]
    gh = jnp.dot(h, wh_ref[...], preferred_element_type=jnp.float32) + bh_ref[...]
    r = jax.nn.sigmoid(gi[:, :D] + gh[:, :D])
    z = jax.nn.sigmoid(gi[:, D:2 * D] + gh[:, D:2 * D])
    n = jnp.tanh(gi[:, 2 * D:] + r * gh[:, 2 * D:])
    out_ref[...] = (1.0 - z) * n + z * h


_gru = pl.pallas_call(
    _gru_body,
    out_shape=jax.ShapeDtypeStruct((B, D), jnp.float32),
    grid=(B // _R,),
    in_specs=[
        pl.BlockSpec((_R, D), lambda i: (i, 0)),
        pl.BlockSpec((_R, D), lambda i: (i, 0)),
        pl.BlockSpec((D, 3 * D), lambda i: (0, 0)),
        pl.BlockSpec((D, 3 * D), lambda i: (0, 0)),
        pl.BlockSpec((1, 3 * D), lambda i: (0, 0)),
        pl.BlockSpec((1, 3 * D), lambda i: (0, 0)),
    ],
    out_specs=pl.BlockSpec((_R, D), lambda i: (i, 0)),
)


_S_NW = _NS
_S_BPW = B // _S_NW
_S_NCH = _S_BPW // _K
_HALF = N_NODES // 2
_C_ROWS = _HALF // _NS
_CROWS = 128
_S_NCOPY = 25
_SLAB = _S_NCOPY * _CROWS
_LU_SZ = 3136
_LLEN = _S_BPW + _K

_scatter_mesh = plsc.VectorSubcoreMesh(core_axis_name="c", subcore_axis_name="s")


@functools.partial(
    pl.kernel,
    out_type=[
        jax.ShapeDtypeStruct((N_NODES, D), jnp.float32),
        jax.ShapeDtypeStruct((N_NODES,), jnp.float32),
    ],
    mesh=_scatter_mesh,
    scratch_types=[
        pltpu.VMEM((_S_BPW,), jnp.int32),
        pltpu.VMEM((_S_BPW,), jnp.int32),
        pltpu.VMEM((_LLEN,), jnp.int32),
        pltpu.VMEM((_LLEN,), jnp.int32),
        pltpu.VMEM((_LLEN // _K, _K), jnp.int32),
        pltpu.VMEM((_CROWS, D), jnp.float32),
        pltpu.VMEM((_CROWS, D), jnp.float32),
        pltpu.VMEM((_LU_SZ,), jnp.float32),
        pltpu.VMEM((_K,), jnp.float32),
        pltpu.SemaphoreType.DMA,
        pltpu.SemaphoreType.DMA,
        pltpu.SemaphoreType.DMA,
    ],
    compiler_params=pltpu.CompilerParams(needs_layout_passes=False),
)
def _sc_scatter(mem_in, lu_in, ids_hbm, aux_hbm, newh_hbm, ts_hbm,
                mem_out, lu_out,
                idin_v, srin_v, dst1, src1, dst2, cbuf0, cbuf1, lubuf, tbuf,
                sem0, sem1, sem2):
    cid = lax.axis_index("c")
    sid = lax.axis_index("s")
    lo = cid * _HALF

    base = sid * _S_BPW
    pltpu.sync_copy(ids_hbm.at[pl.ds(base, _S_BPW)], idin_v)
    for c in range(_S_NCH):
        pltpu.async_copy(aux_hbm.at[idin_v.at[pl.ds(c * _K, _K)]],
                         srin_v.at[pl.ds(c * _K, _K)], sem0)
    for c in range(_S_NCH):
        pltpu.make_async_copy(aux_hbm.at[idin_v.at[pl.ds(c * _K, _K)]],
                              srin_v.at[pl.ds(c * _K, _K)], sem0).wait()

    def _scan_body(i, n):
        ids16 = idin_v[pl.ds(i * 16, 16)]
        src16 = srin_v[pl.ds(i * 16, 16)]
        mine = (ids16 >= lo) & (ids16 < lo + _HALF)
        plsc.store_compressed(dst1.at[pl.ds(n, 16)], ids16, mask=mine)
        plsc.store_compressed(src1.at[pl.ds(n, 16)], src16, mask=mine)
        return n + jnp.sum(jnp.where(mine, 1, 0).astype(jnp.int32))

    n = lax.fori_loop(0, _S_BPW // 16, _scan_body, jnp.int32(0))

    @pl.when(n > 0)
    def _pad():
        sel = jnp.full((16,), n - 1, jnp.int32)
        last_id = plsc.load_gather(dst1, [sel])
        last_src = plsc.load_gather(src1, [sel])
        for k in range(_K // 16):
            dst1[pl.ds(n + k * 16, 16)] = last_id
            src1[pl.ds(n + k * 16, 16)] = last_src

    for r in range(_LLEN // _K):
        for v in range(_K // 16):
            dst2[r, pl.ds(v * 16, 16)] = dst1[pl.ds(r * _K + v * 16, 16)]

    row0 = lo + jnp.minimum((sid * _C_ROWS) // 8 * 8, _HALF - _SLAB)
    bufs = (cbuf0, cbuf1)
    st = [None, None]
    for i in range(_S_NCOPY):
        b = i % 2
        if i >= 2:
            st[b].wait()
        off = row0 + i * _CROWS
        pltpu.async_copy(mem_in.at[pl.ds(off, _CROWS)], bufs[b], sem0).wait()
        st[b] = pltpu.async_copy(bufs[b], mem_out.at[pl.ds(off, _CROWS)], sem1)
    st[(_S_NCOPY - 1) % 2].wait()
    st[(_S_NCOPY - 2) % 2].wait()

    lu0 = lo + jnp.minimum((sid * _C_ROWS) // 8 * 8, _HALF - _LU_SZ)
    pltpu.sync_copy(lu_in.at[pl.ds(lu0, _LU_SZ)], lubuf)
    pltpu.sync_copy(lubuf, lu_out.at[pl.ds(lu0, _LU_SZ)])

    plsc.subcore_barrier()

    nch = (n + (_K - 1)) // _K

    def _chunk_body(c, _):
        srcs = src1.at[pl.ds(c * _K, _K)]
        pltpu.async_copy(newh_hbm.at[srcs], cbuf0, sem0).wait()
        pltpu.async_copy(cbuf0, mem_out.at[dst2.at[c]], sem1).wait()
        pltpu.async_copy(ts_hbm.at[srcs], tbuf, sem2).wait()
        pltpu.async_copy(tbuf, lu_out.at[dst2.at[c]], sem2).wait()
        return _

    lax.fori_loop(0, nch, _chunk_body, jnp.int32(0))


def kernel(unique_node_ids, unique_messages, timestamps, memory, last_update,
           W_ih, W_hh, b_ih, b_hh):
    ids = unique_node_ids
    h, aux = _sc_gather(memory, ids)
    new_h = _gru(unique_messages, h, W_ih.T, W_hh.T,
                 b_ih.reshape(1, 3 * D), b_hh.reshape(1, 3 * D))
    mem_out, lu_out = _sc_scatter(memory, last_update, ids, aux, new_h,
                                  timestamps)
    return mem_out, lu_out


# speedup vs baseline: 1.2527x; 1.2527x over previous
import functools

import jax
import jax.numpy as jnp
from jax import lax
from jax.experimental import pallas as pl
from jax.experimental.pallas import tpu as pltpu
from jax.experimental.pallas import tpu_sc as plsc

N_NODES = 100000
D = 256
B = 16384

_NC = 2
_NS = 16
_NW = _NC * _NS
_K = 128

_G_BPW = B // _NW
_G_NCH = _G_BPW // _K

_gather_mesh = plsc.VectorSubcoreMesh(core_axis_name="c", subcore_axis_name="s")

_A_ROWS = N_NODES // _NW
_A_SLAB = 3200


@functools.partial(
    pl.kernel,
    out_type=[
        jax.ShapeDtypeStruct((B, D), jnp.float32),
        jax.ShapeDtypeStruct((N_NODES,), jnp.int32),
    ],
    mesh=_gather_mesh,
    scratch_types=[
        pltpu.VMEM((_G_NCH, _K), jnp.int32),
        pltpu.VMEM((B,), jnp.int32),
        pltpu.VMEM((_A_SLAB,), jnp.int32),
        pltpu.VMEM((_K, D), jnp.float32),
        pltpu.VMEM((_K, D), jnp.float32),
        pltpu.SemaphoreType.DMA,
        pltpu.SemaphoreType.DMA,
    ],
    compiler_params=pltpu.CompilerParams(needs_layout_passes=False),
)
def _sc_gather(mem_hbm, ids_hbm, out_hbm, aux_hbm,
               idx_v, allids_v, aux_v, buf0, buf1, sem0, sem1):
    wid = lax.axis_index("s") * _NC + lax.axis_index("c")
    base = wid * _G_BPW
    for c in range(_G_NCH):
        pltpu.sync_copy(ids_hbm.at[pl.ds(base + c * _K, _K)], idx_v.at[c])
    pltpu.sync_copy(ids_hbm, allids_v)

    bufs = (buf0, buf1)
    sems = (sem0, sem1)
    cps = [None, None]
    for c in range(_G_NCH):
        cps[c % 2] = pltpu.async_copy(mem_hbm.at[idx_v.at[c]], bufs[c % 2], sems[c % 2])
        if c >= 1:
            cps[(c - 1) % 2].wait()
            pltpu.sync_copy(bufs[(c - 1) % 2], out_hbm.at[pl.ds(base + (c - 1) * _K, _K)])
    cps[(_G_NCH - 1) % 2].wait()
    pltpu.sync_copy(bufs[(_G_NCH - 1) % 2],
                    out_hbm.at[pl.ds(base + (_G_NCH - 1) * _K, _K)])

    alo = jnp.minimum((wid * _A_ROWS) // 8 * 8, N_NODES - _A_SLAB)
    lane = lax.iota(jnp.int32, 16)
    nb_idx = jnp.minimum(lane + 1, 15)

    def _win_body(i, _):
        ids16 = allids_v[pl.ds(i * 16, 16)]
        ivec = lane + i * 16
        ks, iwin = plsc.sort_key_val(ids16 * 16384 + ivec, ivec)
        sid16 = lax.shift_right_logical(ks, 14)
        nb = sid16.at[nb_idx].get(mode=lax.GatherScatterMode.PROMISE_IN_BOUNDS)
        m = ((lane == 15) | (nb != sid16)) & (sid16 >= alo) & (sid16 < alo + _A_SLAB)
        plsc.store_scatter(aux_v, [jnp.where(m, sid16 - alo, 0)], iwin, mask=m)
        return _

    lax.fori_loop(0, B // 16, _win_body, jnp.int32(0))
    pltpu.sync_copy(aux_v, aux_hbm.at[pl.ds(alo, _A_SLAB)])


_R = 1024


def _gru_body(x_ref, h_ref, wi_ref, wh_ref, bi_ref, bh_ref, out_ref):
    h = h_ref[...]
    gi = jnp.dot(x_ref[...], wi_ref[...], preferred_element_type=jnp.float32) + bi_ref[...]
    gh = jnp.dot(h, wh_ref[...], preferred_element_type=jnp.float32) + bh_ref[...]
    r = jax.nn.sigmoid(gi[:, :D] + gh[:, :D])
    z = jax.nn.sigmoid(gi[:, D:2 * D] + gh[:, D:2 * D])
    n = jnp.tanh(gi[:, 2 * D:] + r * gh[:, 2 * D:])
    out_ref[...] = (1.0 - z) * n + z * h


_gru = pl.pallas_call(
    _gru_body,
    out_shape=jax.ShapeDtypeStruct((B, D), jnp.float32),
    grid=(B // _R,),
    in_specs=[
        pl.BlockSpec((_R, D), lambda i: (i, 0)),
        pl.BlockSpec((_R, D), lambda i: (i, 0)),
        pl.BlockSpec((D, 3 * D), lambda i: (0, 0)),
        pl.BlockSpec((D, 3 * D), lambda i: (0, 0)),
        pl.BlockSpec((1, 3 * D), lambda i: (0, 0)),
        pl.BlockSpec((1, 3 * D), lambda i: (0, 0)),
    ],
    out_specs=pl.BlockSpec((_R, D), lambda i: (i, 0)),
)


_S_BPW = B // _NW
_S_NCH = _S_BPW // _K

_scatter_mesh = plsc.VectorSubcoreMesh(core_axis_name="c", subcore_axis_name="s")


@functools.partial(
    pl.kernel,
    out_type=(),
    mesh=_scatter_mesh,
    scratch_types=[
        pltpu.VMEM((_S_NCH, _K), jnp.int32),
        pltpu.VMEM((_S_NCH, _K), jnp.int32),
        pltpu.VMEM((_K, D), jnp.float32),
        pltpu.VMEM((_K, D), jnp.float32),
        pltpu.VMEM((_S_NCH, _K), jnp.float32),
        pltpu.SemaphoreType.DMA,
        pltpu.SemaphoreType.DMA,
        pltpu.SemaphoreType.DMA,
    ],
    compiler_params=pltpu.CompilerParams(needs_layout_passes=False),
)
def _sc_scatter(ids_hbm, aux_hbm, newh_hbm, ts_hbm, mem_ref, lu_ref,
                idx_v, src_v, buf0, buf1, tbuf,
                sem0, sem1, sem2):
    wid = lax.axis_index("s") * _NC + lax.axis_index("c")
    base = wid * _S_BPW
    for c in range(_S_NCH):
        pltpu.sync_copy(ids_hbm.at[pl.ds(base + c * _K, _K)], idx_v.at[c])
    cps = []
    for c in range(_S_NCH):
        cps.append(pltpu.async_copy(aux_hbm.at[idx_v.at[c]], src_v.at[c], sem0))
    for cp in cps:
        cp.wait()
    cps = []
    for c in range(_S_NCH):
        cps.append(pltpu.async_copy(ts_hbm.at[src_v.at[c]], tbuf.at[c], sem2))

    bufs = (buf0, buf1)
    st = [None, None]
    for c in range(_S_NCH):
        b = c % 2
        if c >= 2:
            st[b].wait()
        pltpu.async_copy(newh_hbm.at[src_v.at[c]], bufs[b], sem0).wait()
        st[b] = pltpu.async_copy(bufs[b], mem_ref.at[idx_v.at[c]], sem1)
    for cp in cps:
        cp.wait()
    for c in range(_S_NCH):
        pltpu.sync_copy(tbuf.at[c], lu_ref.at[idx_v.at[c]])
    st[(_S_NCH - 1) % 2].wait()
    st[(_S_NCH - 2) % 2].wait()


def kernel(unique_node_ids, unique_messages, timestamps, memory, last_update,
           W_ih, W_hh, b_ih, b_hh):
    ids = unique_node_ids
    mem_ref = jax.new_ref(memory)
    lu_ref = jax.new_ref(last_update)
    h, aux = _sc_gather(memory, ids)
    new_h = _gru(unique_messages, h, W_ih.T, W_hh.T,
                 b_ih.reshape(1, 3 * D), b_hh.reshape(1, 3 * D))
    _sc_scatter(ids, aux, new_h, timestamps, mem_ref, lu_ref)
    return mem_ref[...], lu_ref[...]


# speedup vs baseline: 1.3123x; 1.0476x over previous
import functools

import jax
import jax.numpy as jnp
from jax import lax
from jax.experimental import pallas as pl
from jax.experimental.pallas import tpu as pltpu
from jax.experimental.pallas import tpu_sc as plsc

N_NODES = 100000
D = 256
B = 16384

_NC = 2
_NS = 16
_NW = _NC * _NS
_K = 128

_G_BPW = B // _NW
_G_NCH = _G_BPW // _K

_gather_mesh = plsc.VectorSubcoreMesh(core_axis_name="c", subcore_axis_name="s")

_A_ROWS = N_NODES // _NW
_A_SLAB = 3200


@functools.partial(
    pl.kernel,
    out_type=[
        jax.ShapeDtypeStruct((B, D), jnp.float32),
        jax.ShapeDtypeStruct((N_NODES,), jnp.int32),
    ],
    mesh=_gather_mesh,
    scratch_types=[
        pltpu.VMEM((_G_NCH, _K), jnp.int32),
        pltpu.VMEM((B,), jnp.int32),
        pltpu.VMEM((_A_SLAB,), jnp.int32),
        pltpu.VMEM((_K, D), jnp.float32),
        pltpu.VMEM((_K, D), jnp.float32),
        pltpu.SemaphoreType.DMA,
        pltpu.SemaphoreType.DMA,
    ],
    compiler_params=pltpu.CompilerParams(needs_layout_passes=False),
)
def _sc_gather(mem_hbm, ids_hbm, out_hbm, aux_hbm,
               idx_v, allids_v, aux_v, buf0, buf1, sem0, sem1):
    wid = lax.axis_index("s") * _NC + lax.axis_index("c")
    base = wid * _G_BPW
    for c in range(_G_NCH):
        pltpu.sync_copy(ids_hbm.at[pl.ds(base + c * _K, _K)], idx_v.at[c])
    pltpu.sync_copy(ids_hbm, allids_v)

    alo = jnp.minimum((wid * _A_ROWS) // 8 * 8, N_NODES - _A_SLAB)
    lane = lax.iota(jnp.int32, 16)
    nb_idx = jnp.minimum(lane + 1, 15)

    def _win_body(i, _):
        ids16 = allids_v[pl.ds(i * 16, 16)]
        ivec = lane + i * 16
        ks, iwin = plsc.sort_key_val(ids16 * 16384 + ivec, ivec)
        sid16 = lax.shift_right_logical(ks, 14)
        nb = sid16.at[nb_idx].get(mode=lax.GatherScatterMode.PROMISE_IN_BOUNDS)
        m = ((lane == 15) | (nb != sid16)) & (sid16 >= alo) & (sid16 < alo + _A_SLAB)
        plsc.store_scatter(aux_v, [jnp.where(m, sid16 - alo, 0)], iwin, mask=m)
        return _

    _SEG = B // 16 // _G_NCH
    bufs = (buf0, buf1)
    ld = [None, None]
    st = [None, None]
    ld[0] = pltpu.async_copy(mem_hbm.at[idx_v.at[0]], bufs[0], sem0)
    for c in range(_G_NCH):
        b = c % 2
        nb = (c + 1) % 2
        if c + 1 < _G_NCH:
            if c + 1 >= 2:
                st[nb].wait()
            ld[nb] = pltpu.async_copy(mem_hbm.at[idx_v.at[c + 1]], bufs[nb], sem0)
        lax.fori_loop(c * _SEG, (c + 1) * _SEG, _win_body, jnp.int32(0))
        ld[b].wait()
        st[b] = pltpu.async_copy(bufs[b], out_hbm.at[pl.ds(base + c * _K, _K)], sem1)
    st[(_G_NCH - 1) % 2].wait()
    st[(_G_NCH - 2) % 2].wait()
    pltpu.sync_copy(aux_v, aux_hbm.at[pl.ds(alo, _A_SLAB)])


_R = 1024


def _gru_body(x_ref, h_ref, wi_ref, wh_ref, bi_ref, bh_ref, out_ref):
    h = h_ref[...]
    gi = jnp.dot(x_ref[...], wi_ref[...], preferred_element_type=jnp.float32) + bi_ref[...]
    gh = jnp.dot(h, wh_ref[...], preferred_element_type=jnp.float32) + bh_ref[...]
    r = jax.nn.sigmoid(gi[:, :D] + gh[:, :D])
    z = jax.nn.sigmoid(gi[:, D:2 * D] + gh[:, D:2 * D])
    n = jnp.tanh(gi[:, 2 * D:] + r * gh[:, 2 * D:])
    out_ref[...] = (1.0 - z) * n + z * h


_gru = pl.pallas_call(
    _gru_body,
    out_shape=jax.ShapeDtypeStruct((B, D), jnp.float32),
    grid=(B // _R,),
    in_specs=[
        pl.BlockSpec((_R, D), lambda i: (i, 0)),
        pl.BlockSpec((_R, D), lambda i: (i, 0)),
        pl.BlockSpec((D, 3 * D), lambda i: (0, 0)),
        pl.BlockSpec((D, 3 * D), lambda i: (0, 0)),
        pl.BlockSpec((1, 3 * D), lambda i: (0, 0)),
        pl.BlockSpec((1, 3 * D), lambda i: (0, 0)),
    ],
    out_specs=pl.BlockSpec((_R, D), lambda i: (i, 0)),
)


_S_BPW = B // _NW
_S_NCH = _S_BPW // _K

_scatter_mesh = plsc.VectorSubcoreMesh(core_axis_name="c", subcore_axis_name="s")


@functools.partial(
    pl.kernel,
    out_type=(),
    mesh=_scatter_mesh,
    scratch_types=[
        pltpu.VMEM((_S_NCH, _K), jnp.int32),
        pltpu.VMEM((_S_NCH, _K), jnp.int32),
        pltpu.VMEM((_K, D), jnp.float32),
        pltpu.VMEM((_K, D), jnp.float32),
        pltpu.VMEM((_S_NCH, _K), jnp.float32),
        pltpu.SemaphoreType.DMA,
        pltpu.SemaphoreType.DMA,
        pltpu.SemaphoreType.DMA,
    ],
    compiler_params=pltpu.CompilerParams(needs_layout_passes=False),
)
def _sc_scatter(ids_hbm, aux_hbm, newh_hbm, ts_hbm, mem_ref, lu_ref,
                idx_v, src_v, buf0, buf1, tbuf,
                sem0, sem1, sem2):
    wid = lax.axis_index("s") * _NC + lax.axis_index("c")
    base = wid * _S_BPW
    for c in range(_S_NCH):
        pltpu.sync_copy(ids_hbm.at[pl.ds(base + c * _K, _K)], idx_v.at[c])
    cps = []
    for c in range(_S_NCH):
        cps.append(pltpu.async_copy(aux_hbm.at[idx_v.at[c]], src_v.at[c], sem0))
    for cp in cps:
        cp.wait()
    cps = []
    for c in range(_S_NCH):
        cps.append(pltpu.async_copy(ts_hbm.at[src_v.at[c]], tbuf.at[c], sem2))

    bufs = (buf0, buf1)
    ld = [None, None]
    st = [None, None]
    ld[0] = pltpu.async_copy(newh_hbm.at[src_v.at[0]], bufs[0], sem0)
    for c in range(_S_NCH):
        b = c % 2
        nb = (c + 1) % 2
        if c + 1 < _S_NCH:
            if c + 1 >= 2:
                st[nb].wait()
            ld[nb] = pltpu.async_copy(newh_hbm.at[src_v.at[c + 1]], bufs[nb], sem0)
        ld[b].wait()
        st[b] = pltpu.async_copy(bufs[b], mem_ref.at[idx_v.at[c]], sem1)
    for cp in cps:
        cp.wait()
    for c in range(_S_NCH):
        pltpu.sync_copy(tbuf.at[c], lu_ref.at[idx_v.at[c]])
    st[(_S_NCH - 1) % 2].wait()
    st[(_S_NCH - 2) % 2].wait()


def kernel(unique_node_ids, unique_messages, timestamps, memory, last_update,
           W_ih, W_hh, b_ih, b_hh):
    ids = unique_node_ids
    mem_ref = jax.new_ref(memory)
    lu_ref = jax.new_ref(last_update)
    h, aux = _sc_gather(memory, ids)
    new_h = _gru(unique_messages, h, W_ih.T, W_hh.T,
                 b_ih.reshape(1, 3 * D), b_hh.reshape(1, 3 * D))
    _sc_scatter(ids, aux, new_h, timestamps, mem_ref, lu_ref)
    return mem_ref[...], lu_ref[...]


# speedup vs baseline: 1.3154x; 1.0024x over previous
import functools

import jax
import jax.numpy as jnp
from jax import lax
from jax.experimental import pallas as pl
from jax.experimental.pallas import tpu as pltpu
from jax.experimental.pallas import tpu_sc as plsc

N_NODES = 100000
D = 256
B = 16384

_NC = 2
_NS = 16
_NW = _NC * _NS
_K = 128

_G_BPW = B // _NW
_G_NCH = _G_BPW // _K

_gather_mesh = plsc.VectorSubcoreMesh(core_axis_name="c", subcore_axis_name="s")

_A_ROWS = N_NODES // _NW
_A_SLAB = 3200


@functools.partial(
    pl.kernel,
    out_type=[
        jax.ShapeDtypeStruct((B, D), jnp.float32),
        jax.ShapeDtypeStruct((N_NODES,), jnp.int32),
    ],
    mesh=_gather_mesh,
    scratch_types=[
        pltpu.VMEM((_G_NCH, _K), jnp.int32),
        pltpu.VMEM((B,), jnp.int32),
        pltpu.VMEM((_A_SLAB,), jnp.int32),
        pltpu.VMEM((_K, D), jnp.float32),
        pltpu.VMEM((_K, D), jnp.float32),
        pltpu.SemaphoreType.DMA,
        pltpu.SemaphoreType.DMA,
    ],
    compiler_params=pltpu.CompilerParams(needs_layout_passes=False),
)
def _sc_gather(mem_hbm, ids_hbm, out_hbm, aux_hbm,
               idx_v, allids_v, aux_v, buf0, buf1, sem0, sem1):
    wid = lax.axis_index("s") * _NC + lax.axis_index("c")
    base = wid * _G_BPW
    for c in range(_G_NCH):
        pltpu.sync_copy(ids_hbm.at[pl.ds(base + c * _K, _K)], idx_v.at[c])
    pltpu.sync_copy(ids_hbm, allids_v)

    alo = jnp.minimum((wid * _A_ROWS) // 8 * 8, N_NODES - _A_SLAB)
    lane = lax.iota(jnp.int32, 16)
    nb_idx = jnp.minimum(lane + 1, 15)

    def _win_one(i):
        ids16 = allids_v[pl.ds(i * 16, 16)]
        ivec = lane + i * 16
        ks, iwin = plsc.sort_key_val(ids16 * 16384 + ivec, ivec)
        sid16 = lax.shift_right_logical(ks, 14)
        nb = sid16.at[nb_idx].get(mode=lax.GatherScatterMode.PROMISE_IN_BOUNDS)
        m = ((lane == 15) | (nb != sid16)) & (sid16 >= alo) & (sid16 < alo + _A_SLAB)
        plsc.store_scatter(aux_v, [jnp.where(m, sid16 - alo, 0)], iwin, mask=m)

    def _win_body(j, _):
        _win_one(j * 2)
        _win_one(j * 2 + 1)
        return _

    _SEG = B // 16 // _G_NCH // 2
    bufs = (buf0, buf1)
    ld = [None, None]
    st = [None, None]
    ld[0] = pltpu.async_copy(mem_hbm.at[idx_v.at[0]], bufs[0], sem0)
    for c in range(_G_NCH):
        b = c % 2
        nb = (c + 1) % 2
        if c + 1 < _G_NCH:
            if c + 1 >= 2:
                st[nb].wait()
            ld[nb] = pltpu.async_copy(mem_hbm.at[idx_v.at[c + 1]], bufs[nb], sem0)
        lax.fori_loop(c * _SEG, (c + 1) * _SEG, _win_body, jnp.int32(0))
        ld[b].wait()
        st[b] = pltpu.async_copy(bufs[b], out_hbm.at[pl.ds(base + c * _K, _K)], sem1)
    st[(_G_NCH - 1) % 2].wait()
    st[(_G_NCH - 2) % 2].wait()
    pltpu.sync_copy(aux_v, aux_hbm.at[pl.ds(alo, _A_SLAB)])


_R = 1024


def _gru_body(x_ref, h_ref, wi_ref, wh_ref, bi_ref, bh_ref, out_ref):
    h = h_ref[...]
    gi = jnp.dot(x_ref[...], wi_ref[...], preferred_element_type=jnp.float32) + bi_ref[...]
    gh = jnp.dot(h, wh_ref[...], preferred_element_type=jnp.float32) + bh_ref[...]
    r = jax.nn.sigmoid(gi[:, :D] + gh[:, :D])
    z = jax.nn.sigmoid(gi[:, D:2 * D] + gh[:, D:2 * D])
    n = jnp.tanh(gi[:, 2 * D:] + r * gh[:, 2 * D:])
    out_ref[...] = (1.0 - z) * n + z * h


_gru = pl.pallas_call(
    _gru_body,
    out_shape=jax.ShapeDtypeStruct((B, D), jnp.float32),
    grid=(B // _R,),
    in_specs=[
        pl.BlockSpec((_R, D), lambda i: (i, 0)),
        pl.BlockSpec((_R, D), lambda i: (i, 0)),
        pl.BlockSpec((D, 3 * D), lambda i: (0, 0)),
        pl.BlockSpec((D, 3 * D), lambda i: (0, 0)),
        pl.BlockSpec((1, 3 * D), lambda i: (0, 0)),
        pl.BlockSpec((1, 3 * D), lambda i: (0, 0)),
    ],
    out_specs=pl.BlockSpec((_R, D), lambda i: (i, 0)),
)


_S_BPW = B // _NW
_S_NCH = _S_BPW // _K

_scatter_mesh = plsc.VectorSubcoreMesh(core_axis_name="c", subcore_axis_name="s")


@functools.partial(
    pl.kernel,
    out_type=(),
    mesh=_scatter_mesh,
    scratch_types=[
        pltpu.VMEM((_S_NCH, _K), jnp.int32),
        pltpu.VMEM((_S_NCH, _K), jnp.int32),
        pltpu.VMEM((_K, D), jnp.float32),
        pltpu.VMEM((_K, D), jnp.float32),
        pltpu.VMEM((_S_NCH, _K), jnp.float32),
        pltpu.SemaphoreType.DMA,
        pltpu.SemaphoreType.DMA,
        pltpu.SemaphoreType.DMA,
    ],
    compiler_params=pltpu.CompilerParams(needs_layout_passes=False),
)
def _sc_scatter(ids_hbm, aux_hbm, newh_hbm, ts_hbm, mem_ref, lu_ref,
                idx_v, src_v, buf0, buf1, tbuf,
                sem0, sem1, sem2):
    wid = lax.axis_index("s") * _NC + lax.axis_index("c")
    base = wid * _S_BPW
    for c in range(_S_NCH):
        pltpu.sync_copy(ids_hbm.at[pl.ds(base + c * _K, _K)], idx_v.at[c])
    cps = []
    for c in range(_S_NCH):
        cps.append(pltpu.async_copy(aux_hbm.at[idx_v.at[c]], src_v.at[c], sem0))
    for cp in cps:
        cp.wait()
    cps = []
    for c in range(_S_NCH):
        cps.append(pltpu.async_copy(ts_hbm.at[src_v.at[c]], tbuf.at[c], sem2))

    bufs = (buf0, buf1)
    ld = [None, None]
    st = [None, None]
    ld[0] = pltpu.async_copy(newh_hbm.at[src_v.at[0]], bufs[0], sem0)
    for c in range(_S_NCH):
        b = c % 2
        nb = (c + 1) % 2
        if c + 1 < _S_NCH:
            if c + 1 >= 2:
                st[nb].wait()
            ld[nb] = pltpu.async_copy(newh_hbm.at[src_v.at[c + 1]], bufs[nb], sem0)
        ld[b].wait()
        st[b] = pltpu.async_copy(bufs[b], mem_ref.at[idx_v.at[c]], sem1)
    for cp in cps:
        cp.wait()
    for c in range(_S_NCH):
        pltpu.sync_copy(tbuf.at[c], lu_ref.at[idx_v.at[c]])
    st[(_S_NCH - 1) % 2].wait()
    st[(_S_NCH - 2) % 2].wait()


def kernel(unique_node_ids, unique_messages, timestamps, memory, last_update,
           W_ih, W_hh, b_ih, b_hh):
    ids = unique_node_ids
    mem_ref = jax.new_ref(memory)
    lu_ref = jax.new_ref(last_update)
    h, aux = _sc_gather(memory, ids)
    new_h = _gru(unique_messages, h, W_ih.T, W_hh.T,
                 b_ih.reshape(1, 3 * D), b_hh.reshape(1, 3 * D))
    _sc_scatter(ids, aux, new_h, timestamps, mem_ref, lu_ref)
    return mem_ref[...], lu_ref[...]
